# baseline (speedup 1.0000x reference)
import functools

import jax
import jax.numpy as jnp
from jax import lax
from jax.experimental import pallas as pl
from jax.experimental.pallas import tpu as pltpu
from jax.experimental.pallas import tpu_sc as plsc

_SCALE = 30.0
_LOG2E = 1.4426950408889634
_LN2 = 0.6931471805599453
_BC = 1024


def _ce_body(tot_cols, x_ref, w_ref, v_ref, out_ref, m_ref, s_ref, xb_ref):
    j = pl.program_id(0)
    ncb = pl.num_programs(0)

    @pl.when(j == 0)
    def _init():
        m_ref[...] = jnp.full(m_ref.shape, -jnp.inf, m_ref.dtype)
        s_ref[...] = jnp.zeros(s_ref.shape, s_ref.dtype)
        xb_ref[...] = x_ref[...].astype(jnp.bfloat16)

    z32 = lax.dot_general(w_ref[...], xb_ref[...], (((1,), (1,)), ((), ())),
                          preferred_element_type=jnp.float32)

    def _update(z32v):
        zz = z32v.astype(jnp.bfloat16)
        m_old = m_ref[...]
        bm = jnp.max(z32v, axis=0, keepdims=True)
        m_new = jnp.maximum(m_old, bm)
        e = jnp.exp2(zz - m_new.astype(jnp.bfloat16))
        ones = jnp.ones((1, e.shape[0]), jnp.bfloat16)
        es = lax.dot_general(ones, e, (((1,), (0,)), ((), ())),
                             preferred_element_type=jnp.float32)
        s_ref[...] = s_ref[...] * jnp.exp2(m_old - m_new) + es
        m_ref[...] = m_new

    @pl.when(j < ncb - 1)
    def _interior():
        _update(z32)

    @pl.when(j == ncb - 1)
    def _fin():
        lim = tot_cols - (ncb - 1) * _BC
        row = lax.broadcasted_iota(jnp.int32, z32.shape, 0)
        _update(jnp.where(row < lim, z32, -jnp.inf))
        lse2 = m_ref[...] + jnp.log(s_ref[...]) * _LOG2E
        out_ref[0, 0] = jnp.sum(lse2 * v_ref[...])


def _ce_call(batch, feat, tot_cols):
    ncb = pl.cdiv(tot_cols, _BC)
    return pl.pallas_call(
        functools.partial(_ce_body, tot_cols),
        grid=(ncb,),
        in_specs=[
            pl.BlockSpec((batch, feat), lambda j: (0, 0)),
            pl.BlockSpec((_BC, feat), lambda j: (j, 0)),
            pl.BlockSpec((1, batch), lambda j: (0, 0)),
        ],
        out_specs=pl.BlockSpec((1, 1), lambda j: (0, 0),
                               memory_space=pltpu.SMEM),
        out_shape=jax.ShapeDtypeStruct((1, 1), jnp.float32),
        scratch_shapes=[
            pltpu.VMEM((1, batch), jnp.float32),
            pltpu.VMEM((1, batch), jnp.float32),
            pltpu.VMEM((batch, feat), jnp.bfloat16),
        ],
    )


def _fin_body(x_ref, g_ref, v_ref, a_ref, out_ref):
    p = (x_ref[...] * g_ref[...]).astype(jnp.bfloat16)
    vb = v_ref[...].astype(jnp.bfloat16)
    ts = lax.dot_general(vb, p, (((1,), (0,)), ((), ())),
                         preferred_element_type=jnp.float32)
    bsum = jnp.sum(ts)
    den = jnp.maximum(jnp.sum(v_ref[...]), 1.0)
    out_ref[0, 0] = (_LN2 * a_ref[0, 0] - _SCALE * bsum) / den


def _fin_call(batch, feat):
    return pl.pallas_call(
        _fin_body,
        in_specs=[
            pl.BlockSpec((batch, feat), lambda: (0, 0)),
            pl.BlockSpec((batch, feat), lambda: (0, 0)),
            pl.BlockSpec((1, batch), lambda: (0, 0)),
            pl.BlockSpec((1, 1), lambda: (0, 0), memory_space=pltpu.SMEM),
        ],
        out_specs=pl.BlockSpec((1, 1), lambda: (0, 0),
                               memory_space=pltpu.SMEM),
        out_shape=jax.ShapeDtypeStruct((1, 1), jnp.float32),
    )


@functools.lru_cache
def _sc_gather(num_rows, feat, batch):
    info = plsc.get_sparse_core_info()
    nw = info.num_cores * info.num_subcores
    bpw = batch // nw
    mesh = plsc.VectorSubcoreMesh(core_axis_name="c", subcore_axis_name="s")

    @functools.partial(
        pl.kernel, mesh=mesh,
        out_type=jax.ShapeDtypeStruct((batch, feat), jnp.float32),
        scratch_types=[
            pltpu.VMEM((bpw,), jnp.int32),
            pltpu.VMEM((bpw, feat), jnp.float32),
            pltpu.SemaphoreType.DMA,
        ],
    )
    def gk(table_hbm, idx_hbm, out_hbm, idx_v, rows_v, sem):
        wid = lax.axis_index("s") * info.num_cores + lax.axis_index("c")
        base = wid * bpw
        pltpu.sync_copy(idx_hbm.at[pl.ds(base, bpw)], idx_v)
        pltpu.async_copy(table_hbm.at[idx_v], rows_v, sem).wait()
        pltpu.sync_copy(rows_v, out_hbm.at[pl.ds(base, bpw)])

    return gk


def kernel(inputs, roi_label, lut, cq, cq_omega):
    batch, feat = inputs.shape
    tot_cols = lut.shape[0] + cq.shape[0]

    lab = roi_label.reshape(-1).astype(jnp.int32) - 1
    validf = (lab >= 0).astype(jnp.float32).reshape(1, batch)
    safe = jnp.maximum(lab, 0)

    g = _sc_gather(lut.shape[0], feat, batch)(lut, safe)

    w = (jnp.concatenate([lut, cq], axis=0) * (_SCALE * _LOG2E)
         ).astype(jnp.bfloat16)
    a = _ce_call(batch, feat, tot_cols)(inputs, w, validf)
    out = _fin_call(batch, feat)(inputs, g, validf, a)
    return out[0, 0]


# speedup vs baseline: 1.0328x; 1.0328x over previous
import functools

import jax
import jax.numpy as jnp
from jax import lax
from jax.experimental import pallas as pl
from jax.experimental.pallas import tpu as pltpu
from jax.experimental.pallas import tpu_sc as plsc

_SCALE = 30.0
_LOG2E = 1.4426950408889634
_LN2 = 0.6931471805599453
_BC = 1024


def _ce_body(tot_cols, x_ref, w_ref, v_ref, out_ref, m_ref, s_ref, xb_ref):
    j = pl.program_id(0)
    ncb = pl.num_programs(0)

    @pl.when(j == 0)
    def _init():
        m_ref[...] = jnp.full(m_ref.shape, -jnp.inf, m_ref.dtype)
        s_ref[...] = jnp.zeros(s_ref.shape, s_ref.dtype)
        xb_ref[...] = x_ref[...].astype(jnp.bfloat16)

    z = lax.dot_general(w_ref[...], xb_ref[...], (((1,), (1,)), ((), ())),
                        preferred_element_type=jnp.float32
                        ).astype(jnp.bfloat16)

    def _update(zz):
        m_old = m_ref[...]
        bm = jnp.max(zz, axis=0, keepdims=True).astype(jnp.float32)
        m_new = jnp.maximum(m_old, bm)
        e = jnp.exp2(zz - m_new.astype(jnp.bfloat16))
        ones = jnp.ones((1, e.shape[0]), jnp.bfloat16)
        es = lax.dot_general(ones, e, (((1,), (0,)), ((), ())),
                             preferred_element_type=jnp.float32)
        s_ref[...] = s_ref[...] * jnp.exp2(m_old - m_new) + es
        m_ref[...] = m_new

    @pl.when(j < ncb - 1)
    def _interior():
        _update(z)

    @pl.when(j == ncb - 1)
    def _fin():
        lim = tot_cols - (ncb - 1) * _BC
        row = lax.broadcasted_iota(jnp.int32, z.shape, 0)
        _update(jnp.where(row < lim, z, -jnp.inf))
        lse2 = m_ref[...] + jnp.log(s_ref[...]) * _LOG2E
        out_ref[0, 0] = jnp.sum(lse2 * v_ref[...])


def _ce_call(batch, feat, tot_cols):
    ncb = pl.cdiv(tot_cols, _BC)
    return pl.pallas_call(
        functools.partial(_ce_body, tot_cols),
        grid=(ncb,),
        in_specs=[
            pl.BlockSpec((batch, feat), lambda j: (0, 0)),
            pl.BlockSpec((_BC, feat), lambda j: (j, 0)),
            pl.BlockSpec((1, batch), lambda j: (0, 0)),
        ],
        out_specs=pl.BlockSpec((1, 1), lambda j: (0, 0),
                               memory_space=pltpu.SMEM),
        out_shape=jax.ShapeDtypeStruct((1, 1), jnp.float32),
        scratch_shapes=[
            pltpu.VMEM((1, batch), jnp.float32),
            pltpu.VMEM((1, batch), jnp.float32),
            pltpu.VMEM((batch, feat), jnp.bfloat16),
        ],
    )


def _fin_body(x_ref, g_ref, v_ref, a_ref, out_ref):
    p = (x_ref[...] * g_ref[...]).astype(jnp.bfloat16)
    vb = v_ref[...].astype(jnp.bfloat16)
    ts = lax.dot_general(vb, p, (((1,), (0,)), ((), ())),
                         preferred_element_type=jnp.float32)
    bsum = jnp.sum(ts)
    den = jnp.maximum(jnp.sum(v_ref[...]), 1.0)
    out_ref[0, 0] = (_LN2 * a_ref[0, 0] - _SCALE * bsum) / den


def _fin_call(batch, feat):
    return pl.pallas_call(
        _fin_body,
        in_specs=[
            pl.BlockSpec((batch, feat), lambda: (0, 0)),
            pl.BlockSpec((batch, feat), lambda: (0, 0)),
            pl.BlockSpec((1, batch), lambda: (0, 0)),
            pl.BlockSpec((1, 1), lambda: (0, 0), memory_space=pltpu.SMEM),
        ],
        out_specs=pl.BlockSpec((1, 1), lambda: (0, 0),
                               memory_space=pltpu.SMEM),
        out_shape=jax.ShapeDtypeStruct((1, 1), jnp.float32),
    )


@functools.lru_cache
def _sc_gather(num_rows, feat, batch):
    info = plsc.get_sparse_core_info()
    nw = info.num_cores * info.num_subcores
    bpw = batch // nw
    mesh = plsc.VectorSubcoreMesh(core_axis_name="c", subcore_axis_name="s")

    @functools.partial(
        pl.kernel, mesh=mesh,
        out_type=jax.ShapeDtypeStruct((batch, feat), jnp.float32),
        scratch_types=[
            pltpu.VMEM((bpw,), jnp.int32),
            pltpu.VMEM((bpw, feat), jnp.float32),
            pltpu.SemaphoreType.DMA,
        ],
    )
    def gk(table_hbm, idx_hbm, out_hbm, idx_v, rows_v, sem):
        wid = lax.axis_index("s") * info.num_cores + lax.axis_index("c")
        base = wid * bpw
        pltpu.sync_copy(idx_hbm.at[pl.ds(base, bpw)], idx_v)
        pltpu.async_copy(table_hbm.at[idx_v], rows_v, sem).wait()
        pltpu.sync_copy(rows_v, out_hbm.at[pl.ds(base, bpw)])

    return gk


def kernel(inputs, roi_label, lut, cq, cq_omega):
    batch, feat = inputs.shape
    tot_cols = lut.shape[0] + cq.shape[0]

    lab = roi_label.reshape(-1).astype(jnp.int32) - 1
    validf = (lab >= 0).astype(jnp.float32).reshape(1, batch)
    safe = jnp.maximum(lab, 0)

    g = _sc_gather(lut.shape[0], feat, batch)(lut, safe)

    w = (jnp.concatenate([lut, cq], axis=0) * (_SCALE * _LOG2E)
         ).astype(jnp.bfloat16)
    a = _ce_call(batch, feat, tot_cols)(inputs, w, validf)
    out = _fin_call(batch, feat)(inputs, g, validf, a)
    return out[0, 0]
